# Initial kernel scaffold; baseline (speedup 1.0000x reference)
import jax
import jax.numpy as jnp
from jax.experimental import pallas as pl


def kernel(x, edge_index, W1, b1, W2, b2, W3, b3):
    raise NotImplementedError("write your pallas kernel here")


# speedup vs baseline: 7.5665x; 7.5665x over previous
import functools
import jax
import jax.numpy as jnp
from jax import lax
from jax.experimental import pallas as pl
from jax.experimental.pallas import tpu as pltpu
from jax.experimental.pallas import tpu_sc as plsc

N = 10000
D = 128
NC = 2
NS = 16
NW = NC * NS
CHUNK = 128
NP = 10240
RPT = NP // NS
ROWBLK = 1024
GRID = NP // ROWBLK

_mesh = plsc.VectorSubcoreMesh(core_axis_name="c", subcore_axis_name="s")


def _make_deg_kernel(nchunk):
  DW = 16

  @functools.partial(
      pl.kernel,
      out_type=jax.ShapeDtypeStruct((NC, NP, DW), jnp.float32),
      mesh=_mesh,
      scratch_types=[
          pltpu.VMEM_SHARED((NP, DW), jnp.float32),
          pltpu.VMEM((nchunk, CHUNK), jnp.int32),
          pltpu.VMEM((CHUNK, DW), jnp.float32),
      ],
  )
  def deg_kernel(dst_hbm, out_hbm, dacc, idx_v, ones_v):
    cid = lax.axis_index("c")
    sid = lax.axis_index("s")
    wid = sid * NC + cid

    zero16 = jnp.zeros((16,), jnp.float32)
    one16 = jnp.ones((16,), jnp.float32)

    @pl.loop(0, CHUNK)
    def _z(r):
      ones_v[r, pl.ds(0, DW)] = zero16

    @pl.loop(0, RPT // CHUNK)
    def _zacc(k):
      pltpu.sync_copy(ones_v, dacc.at[pl.ds(sid * RPT + k * CHUNK, CHUNK)])

    @pl.loop(0, CHUNK)
    def _f(r):
      ones_v[r, pl.ds(0, DW)] = one16

    pltpu.sync_copy(dst_hbm.at[wid], idx_v)
    plsc.subcore_barrier()

    @pl.loop(0, nchunk)
    def _j(j):
      pltpu.sync_copy(ones_v, dacc.at[idx_v.at[j]], add=True)

    plsc.subcore_barrier()
    pltpu.sync_copy(dacc.at[pl.ds(sid * RPT, RPT)],
                    out_hbm.at[cid, pl.ds(sid * RPT, RPT)])

  return deg_kernel


def _make_prop_kernel(nchunk):
  @functools.partial(
      pl.kernel,
      out_type=jax.ShapeDtypeStruct((NC, NP, D), jnp.float32),
      mesh=_mesh,
      scratch_types=[
          pltpu.VMEM_SHARED((NP, D), jnp.float32),
          pltpu.VMEM((nchunk, CHUNK), jnp.int32),
          pltpu.VMEM((nchunk, CHUNK), jnp.int32),
          pltpu.VMEM((CHUNK, D), jnp.float32),
          pltpu.SemaphoreType.DMA,
      ],
  )
  def prop_kernel(hs_hbm, src_hbm, dst_hbm, out_hbm,
                  acc, is_v, id_v, rows_a, sem_a):
    cid = lax.axis_index("c")
    sid = lax.axis_index("s")
    wid = sid * NC + cid

    zero16 = jnp.zeros((16,), jnp.float32)

    @pl.loop(0, CHUNK)
    def _zr(r):
      @pl.loop(0, D // 16)
      def _zc(c):
        rows_a[r, pl.ds(c * 16, 16)] = zero16

    @pl.loop(0, RPT // CHUNK)
    def _zacc(k):
      pltpu.sync_copy(rows_a, acc.at[pl.ds(sid * RPT + k * CHUNK, CHUNK)])

    pltpu.sync_copy(src_hbm.at[wid], is_v)
    pltpu.sync_copy(dst_hbm.at[wid], id_v)
    plsc.subcore_barrier()

    @pl.loop(0, nchunk)
    def _chunk(j):
      pltpu.async_copy(hs_hbm.at[is_v.at[j]], rows_a, sem_a).wait()
      pltpu.sync_copy(rows_a, acc.at[id_v.at[j]], add=True)

    plsc.subcore_barrier()
    pltpu.sync_copy(acc.at[pl.ds(sid * RPT, RPT)],
                    out_hbm.at[cid, pl.ds(sid * RPT, RPT)])

  return prop_kernel


def _tc_first_body(x_ref, w_ref, dinv_ref, o_ref):
  h = jnp.dot(x_ref[...], w_ref[...], preferred_element_type=jnp.float32)
  o_ref[...] = h * dinv_ref[...]


def _tc_mid_body(p_ref, hp_ref, dinv_ref, b_ref, w_ref, o_ref):
  agg = p_ref[0] + p_ref[1] + hp_ref[...]
  xn = jnp.maximum(agg * dinv_ref[...] + b_ref[...], 0.0)
  h = jnp.dot(xn, w_ref[...], preferred_element_type=jnp.float32)
  o_ref[...] = h * dinv_ref[...]


def _tc_last_body(p_ref, hp_ref, dinv_ref, b_ref, o_ref):
  agg = p_ref[0] + p_ref[1] + hp_ref[...]
  o_ref[...] = agg * dinv_ref[...] + b_ref[...]


_rowspec = pl.BlockSpec((ROWBLK, D), lambda i: (i, 0))
_fullspec = pl.BlockSpec((D, D), lambda i: (0, 0))
_bspec = pl.BlockSpec((1, D), lambda i: (0, 0))
_pspec = pl.BlockSpec((NC, ROWBLK, D), lambda i: (0, i, 0))
_oshape = jax.ShapeDtypeStruct((NP, D), jnp.float32)


def _tc_first(xp, W, dinv_b):
  return pl.pallas_call(
      _tc_first_body, grid=(GRID,),
      in_specs=[_rowspec, _fullspec, _rowspec],
      out_specs=_rowspec, out_shape=_oshape,
  )(xp, W, dinv_b)


def _tc_mid(p, hp, dinv_b, b, W):
  return pl.pallas_call(
      _tc_mid_body, grid=(GRID,),
      in_specs=[_pspec, _rowspec, _rowspec, _bspec, _fullspec],
      out_specs=_rowspec, out_shape=_oshape,
  )(p, hp, dinv_b, b.reshape(1, D), W)


def _tc_last(p, hp, dinv_b, b):
  return pl.pallas_call(
      _tc_last_body, grid=(GRID,),
      in_specs=[_pspec, _rowspec, _rowspec, _bspec],
      out_specs=_rowspec, out_shape=_oshape,
  )(p, hp, dinv_b, b.reshape(1, D))


@jax.jit
def kernel(x, edge_index, W1, b1, W2, b2, W3, b3):
  E = edge_index.shape[1]
  nchunk = -(-E // (NW * CHUNK))
  nchunk += nchunk % 2
  e_pad = NW * nchunk * CHUNK - E

  src = jnp.concatenate(
      [edge_index[0].astype(jnp.int32), jnp.zeros((e_pad,), jnp.int32)])
  dst = jnp.concatenate(
      [edge_index[1].astype(jnp.int32), jnp.full((e_pad,), N, jnp.int32)])
  src3 = src.reshape(NW, nchunk, CHUNK)
  dst3 = dst.reshape(NW, nchunk, CHUNK)
  xp = jnp.pad(x, ((0, NP - N), (0, 0)))

  degp = _make_deg_kernel(nchunk)(dst3)
  dinv = lax.rsqrt(degp[0, :, 0] + degp[1, :, 0] + 1.0)
  dinv_b = jnp.broadcast_to(dinv[:, None], (NP, D))

  prop = _make_prop_kernel(nchunk)

  hs1 = _tc_first(xp, W1, dinv_b)
  p1 = prop(hs1, src3, dst3)
  hs2 = _tc_mid(p1, hs1, dinv_b, b1, W2)
  p2 = prop(hs2, src3, dst3)
  hs3 = _tc_mid(p2, hs2, dinv_b, b2, W3)
  p3 = prop(hs3, src3, dst3)
  out = _tc_last(p3, hs3, dinv_b, b3)
  return out[:N]
